# baseline (speedup 1.0000x reference)
import functools

import jax
import jax.numpy as jnp
from jax import lax
from jax.experimental import pallas as pl
from jax.experimental.pallas import tpu as pltpu
from jax.experimental.pallas import tpu_sc as plsc

_NC = 2
_NS = 16
_L = 16
_NW = _NC * _NS


@functools.lru_cache(maxsize=None)
def _make_sc_gather(B, N, E):
    F = N * N
    assert B % _NW == 0 and E % _L == 0
    bpw = B // _NW
    chunks = E // _L
    mesh = plsc.VectorSubcoreMesh(core_axis_name="c", subcore_axis_name="s")

    @functools.partial(
        pl.kernel,
        out_type=jax.ShapeDtypeStruct((B, E), jnp.float32),
        mesh=mesh,
        compiler_params=pltpu.CompilerParams(needs_layout_passes=False),
        scratch_types=[
            pltpu.VMEM((E,), jnp.int32),
            pltpu.VMEM((E,), jnp.int32),
            pltpu.VMEM((E,), jnp.int32),
            pltpu.VMEM((F,), jnp.float32),
            pltpu.VMEM((E,), jnp.float32),
        ],
    )
    def k(x_hbm, rows_hbm, cols_hbm, out_hbm, idx_v, rows_v, cols_v, xv, ov):
        wid = lax.axis_index("s") * _NC + lax.axis_index("c")
        base = wid * bpw

        pltpu.sync_copy(rows_hbm, rows_v)
        pltpu.sync_copy(cols_hbm, cols_v)

        def idx_body(m, carry):
            sl = pl.ds(m * _L, _L)
            idx_v[sl] = rows_v[sl] * N + cols_v[sl]
            return carry

        lax.fori_loop(0, chunks, idx_body, 0)

        def batch_body(b, carry):
            pltpu.sync_copy(x_hbm.at[base + b], xv)

            def gbody(m, c2):
                sl = pl.ds(m * _L, _L)
                ov[sl] = plsc.load_gather(xv, [idx_v[sl]])
                return c2

            lax.fori_loop(0, chunks, gbody, 0)
            pltpu.sync_copy(ov, out_hbm.at[base + b])
            return carry

        lax.fori_loop(0, bpw, batch_body, 0)

    return k


def kernel(x, rows, cols):
    B, N, _ = x.shape
    xflat = x.reshape(B, N * N)
    k = _make_sc_gather(B, N, rows.shape[0])
    return k(xflat, rows.astype(jnp.int32), cols.astype(jnp.int32))


# speedup vs baseline: 1.5727x; 1.5727x over previous
import functools

import jax
import jax.numpy as jnp
from jax import lax
from jax.experimental import pallas as pl
from jax.experimental.pallas import tpu as pltpu
from jax.experimental.pallas import tpu_sc as plsc

_NC = 2
_NS = 16
_L = 16
_NW = _NC * _NS
_UNROLL = 4


@functools.lru_cache(maxsize=None)
def _make_sc_gather(B, N, E):
    F = N * N
    assert B % (2 * _NW) == 0 and E % (_L * _UNROLL) == 0
    bpw = B // _NW
    chunks = E // _L
    mesh = plsc.VectorSubcoreMesh(core_axis_name="c", subcore_axis_name="s")

    @functools.partial(
        pl.kernel,
        out_type=jax.ShapeDtypeStruct((B, E), jnp.float32),
        mesh=mesh,
        compiler_params=pltpu.CompilerParams(needs_layout_passes=False),
        scratch_types=[
            pltpu.VMEM((E,), jnp.int32),
            pltpu.VMEM((E,), jnp.int32),
            pltpu.VMEM((E,), jnp.int32),
            pltpu.VMEM((F,), jnp.float32),
            pltpu.VMEM((F,), jnp.float32),
            pltpu.VMEM((E,), jnp.float32),
            pltpu.VMEM((E,), jnp.float32),
            pltpu.SemaphoreType.DMA,
            pltpu.SemaphoreType.DMA,
            pltpu.SemaphoreType.DMA,
            pltpu.SemaphoreType.DMA,
        ],
    )
    def k(x_hbm, rows_hbm, cols_hbm, out_hbm,
          idx_v, rows_v, cols_v, xva, xvb, ova, ovb,
          sia, sib, soa, sob):
        wid = lax.axis_index("s") * _NC + lax.axis_index("c")
        base = wid * bpw

        pltpu.sync_copy(rows_hbm, rows_v)
        pltpu.sync_copy(cols_hbm, cols_v)

        def idx_body(m, carry):
            sl = pl.ds(m * _L, _L)
            idx_v[sl] = rows_v[sl] * N + cols_v[sl]
            return carry

        lax.fori_loop(0, chunks, idx_body, 0)

        def gather(xv, ov):
            def gbody(m, carry):
                for u in range(_UNROLL):
                    sl = pl.ds((m * _UNROLL + u) * _L, _L)
                    ov[sl] = plsc.load_gather(xv, [idx_v[sl]])
                return carry

            lax.fori_loop(0, chunks // _UNROLL, gbody, 0)

        def step(k_, b, xv, ov, si, so):
            pltpu.make_async_copy(x_hbm.at[b], xv, si).wait()

            @pl.when(k_ > 0)
            def _():
                pltpu.make_async_copy(ov, out_hbm.at[b], so).wait()

            gather(xv, ov)
            pltpu.async_copy(ov, out_hbm.at[b], so)

            @pl.when(b + 2 < base + bpw)
            def _():
                pltpu.async_copy(x_hbm.at[b + 2], xv, si)

        pltpu.async_copy(x_hbm.at[base], xva, sia)
        pltpu.async_copy(x_hbm.at[base + 1], xvb, sib)

        def batch_body(k_, carry):
            step(k_, base + 2 * k_, xva, ova, sia, soa)
            step(k_, base + 2 * k_ + 1, xvb, ovb, sib, sob)
            return carry

        lax.fori_loop(0, bpw // 2, batch_body, 0)
        pltpu.make_async_copy(ova, out_hbm.at[base], soa).wait()
        pltpu.make_async_copy(ovb, out_hbm.at[base], sob).wait()

    return k


def kernel(x, rows, cols):
    B, N, _ = x.shape
    xflat = x.reshape(B, N * N)
    k = _make_sc_gather(B, N, rows.shape[0])
    return k(xflat, rows.astype(jnp.int32), cols.astype(jnp.int32))


# speedup vs baseline: 1.9693x; 1.2522x over previous
import functools

import jax
import jax.numpy as jnp
from jax import lax
from jax.experimental import pallas as pl
from jax.experimental.pallas import tpu as pltpu
from jax.experimental.pallas import tpu_sc as plsc

_NC = 2
_NS = 16
_L = 16
_NW = _NC * _NS
_UNROLL = 4


@functools.lru_cache(maxsize=None)
def _make_sc_gather(B, N, E):
    F = N * N
    assert B % (2 * _NW) == 0 and E % (_L * _UNROLL) == 0
    bpw = B // _NW
    chunks = E // _L
    mesh = plsc.VectorSubcoreMesh(core_axis_name="c", subcore_axis_name="s")

    @functools.partial(
        pl.kernel,
        out_type=jax.ShapeDtypeStruct((B, E), jnp.float32),
        mesh=mesh,
        compiler_params=pltpu.CompilerParams(needs_layout_passes=False),
        scratch_types=[
            pltpu.VMEM((E,), jnp.int32),
            pltpu.VMEM((E,), jnp.int32),
            pltpu.VMEM((E,), jnp.int32),
            pltpu.VMEM((F,), jnp.float32),
            pltpu.VMEM((F,), jnp.float32),
            pltpu.VMEM((E,), jnp.float32),
            pltpu.VMEM((E,), jnp.float32),
            pltpu.SemaphoreType.DMA,
            pltpu.SemaphoreType.DMA,
            pltpu.SemaphoreType.DMA,
            pltpu.SemaphoreType.DMA,
        ],
    )
    def k(x_hbm, rows_hbm, cols_hbm, out_hbm,
          idx_v, rows_v, cols_v, xva, xvb, ova, ovb,
          sia, sib, soa, sob):
        wid = lax.axis_index("s") * _NC + lax.axis_index("c")
        base = wid * bpw

        pltpu.sync_copy(rows_hbm, rows_v)
        pltpu.sync_copy(cols_hbm, cols_v)

        @plsc.parallel_loop(0, chunks, 1, unroll=_UNROLL)
        def _(m):
            sl = pl.ds(m * _L, _L)
            idx_v[sl] = rows_v[sl] * N + cols_v[sl]

        def gather(xv, ov):
            @plsc.parallel_loop(0, chunks, 1, unroll=_UNROLL)
            def _(m):
                sl = pl.ds(m * _L, _L)
                ov[sl] = plsc.load_gather(xv, [idx_v[sl]])

        def step(k_, b, xv, ov, si, so):
            pltpu.make_async_copy(x_hbm.at[b], xv, si).wait()

            @pl.when(k_ > 0)
            def _():
                pltpu.make_async_copy(ov, out_hbm.at[b], so).wait()

            gather(xv, ov)
            pltpu.async_copy(ov, out_hbm.at[b], so)

            @pl.when(b + 2 < base + bpw)
            def _():
                pltpu.async_copy(x_hbm.at[b + 2], xv, si)

        pltpu.async_copy(x_hbm.at[base], xva, sia)
        pltpu.async_copy(x_hbm.at[base + 1], xvb, sib)

        def batch_body(k_, carry):
            step(k_, base + 2 * k_, xva, ova, sia, soa)
            step(k_, base + 2 * k_ + 1, xvb, ovb, sib, sob)
            return carry

        lax.fori_loop(0, bpw // 2, batch_body, 0)
        pltpu.make_async_copy(ova, out_hbm.at[base], soa).wait()
        pltpu.make_async_copy(ovb, out_hbm.at[base], sob).wait()

    return k


def kernel(x, rows, cols):
    B, N, _ = x.shape
    xflat = x.reshape(B, N * N)
    k = _make_sc_gather(B, N, rows.shape[0])
    return k(xflat, rows.astype(jnp.int32), cols.astype(jnp.int32))


# speedup vs baseline: 3.1418x; 1.5954x over previous
import functools

import jax
import jax.numpy as jnp
from jax import lax
from jax.experimental import pallas as pl
from jax.experimental.pallas import tpu as pltpu
from jax.experimental.pallas import tpu_sc as plsc

_NC = 2
_NS = 16
_L = 16
_NW = _NC * _NS
_UNROLL = 4


@functools.lru_cache(maxsize=None)
def _make_sc_gather(B, N, E):
    assert B % (2 * _NW) == 0 and E % (_L * _UNROLL) == 0
    bpw = B // _NW
    chunks = E // _L
    mesh = plsc.VectorSubcoreMesh(core_axis_name="c", subcore_axis_name="s")

    @functools.partial(
        pl.kernel,
        out_type=jax.ShapeDtypeStruct((B, E), jnp.float32),
        mesh=mesh,
        compiler_params=pltpu.CompilerParams(needs_layout_passes=False),
        scratch_types=[
            pltpu.VMEM((E,), jnp.int32),
            pltpu.VMEM((E,), jnp.int32),
            pltpu.VMEM((N, N), jnp.float32),
            pltpu.VMEM((N, N), jnp.float32),
            pltpu.VMEM((E,), jnp.float32),
            pltpu.VMEM((E,), jnp.float32),
            pltpu.SemaphoreType.DMA,
            pltpu.SemaphoreType.DMA,
            pltpu.SemaphoreType.DMA,
            pltpu.SemaphoreType.DMA,
        ],
    )
    def k(x_hbm, rows_hbm, cols_hbm, out_hbm,
          rows_v, cols_v, xva, xvb, ova, ovb,
          sia, sib, soa, sob):
        wid = lax.axis_index("s") * _NC + lax.axis_index("c")
        base = wid * bpw

        pltpu.sync_copy(rows_hbm, rows_v)
        pltpu.sync_copy(cols_hbm, cols_v)

        def gather(xv, ov):
            @plsc.parallel_loop(0, chunks, 1, unroll=_UNROLL)
            def _(m):
                sl = pl.ds(m * _L, _L)
                ov[sl] = plsc.load_gather(xv, [rows_v[sl], cols_v[sl]])

        def step(k_, b, xv, ov, si, so):
            pltpu.make_async_copy(x_hbm.at[b], xv, si).wait()

            @pl.when(k_ > 0)
            def _():
                pltpu.make_async_copy(ov, out_hbm.at[b], so).wait()

            gather(xv, ov)
            pltpu.async_copy(ov, out_hbm.at[b], so)

            @pl.when(b + 2 < base + bpw)
            def _():
                pltpu.async_copy(x_hbm.at[b + 2], xv, si)

        pltpu.async_copy(x_hbm.at[base], xva, sia)
        pltpu.async_copy(x_hbm.at[base + 1], xvb, sib)

        def batch_body(k_, carry):
            step(k_, base + 2 * k_, xva, ova, sia, soa)
            step(k_, base + 2 * k_ + 1, xvb, ovb, sib, sob)
            return carry

        lax.fori_loop(0, bpw // 2, batch_body, 0)
        pltpu.make_async_copy(ova, out_hbm.at[base], soa).wait()
        pltpu.make_async_copy(ovb, out_hbm.at[base], sob).wait()

    return k


def kernel(x, rows, cols):
    B, N, _ = x.shape
    k = _make_sc_gather(B, N, rows.shape[0])
    return k(x, rows.astype(jnp.int32), cols.astype(jnp.int32))
